# baseline (speedup 1.0000x reference)
import jax
import jax.numpy as jnp
from jax import lax
from jax.experimental import pallas as pl
from jax.experimental.pallas import tpu as pltpu

_ROWS = 256
_SPLIT = 2


def _row_threshold(a, k):
    rows = a.shape[0]
    kf = k.astype(jnp.float32)
    parts = [a[i * (rows // _SPLIT):(i + 1) * (rows // _SPLIT)]
             for i in range(_SPLIT)]

    ones_col = jnp.ones((a.shape[1], 1), jnp.float32)

    def bit_step(i, vs):
        bit = jnp.left_shift(jnp.uint32(1), jnp.uint32(30) - i.astype(jnp.uint32))
        out = []
        for ap, v in zip(parts, vs):
            t = v | bit
            tf = lax.bitcast_convert_type(t, jnp.float32)
            cnt = jnp.dot(jnp.where(ap >= tf, 1.0, 0.0), ones_col,
                          preferred_element_type=jnp.float32,
                          precision=lax.Precision.DEFAULT)
            out.append(jnp.where(cnt > kf, t, v))
        return tuple(out)

    v0 = tuple(jnp.zeros((rows // _SPLIT, 1), jnp.uint32) for _ in range(_SPLIT))
    vs = lax.fori_loop(0, 31, bit_step, v0)
    v = jnp.concatenate(vs, axis=0)
    return lax.bitcast_convert_type(v, jnp.float32)


def _body(k_ref, x_ref, w_ref, b1_ref, b2_ref, enc_ref, dec_ref, nnz_ref,
          res_ref):
    enc = jnp.dot(x_ref[...], w_ref[...],
                  preferred_element_type=jnp.float32,
                  precision=lax.Precision.DEFAULT) + b1_ref[...]
    enc_ref[...] = enc
    a = jnp.abs(enc)
    k = k_ref[0]

    thr = _row_threshold(a, k)
    keep = a > thr
    res = jnp.where(keep, enc, 0.0)
    res_ref[...] = res

    @pl.when(pl.program_id(0) == 0)
    def _init():
        nnz_ref[...] = jnp.zeros_like(nnz_ref)

    cnt2 = jnp.sum(jnp.where(keep, 1.0, 0.0), axis=1, keepdims=True)
    nnz_ref[...] += jnp.sum(cnt2, axis=0, keepdims=True)

    dec = lax.dot_general(res, w_ref[...], (((1,), (1,)), ((), ())),
                          preferred_element_type=jnp.float32,
                          precision=lax.Precision.DEFAULT) + b2_ref[...]
    dec_ref[...] = dec


def kernel(x, K, W, b1, b2):
    B, D = x.shape
    m = W.shape[1]
    rows = _ROWS if B % _ROWS == 0 else B
    grid = (B // rows,)
    k_arr = jnp.asarray(K, jnp.int32).reshape(1)
    enc, dec, nnz, res = pl.pallas_call(
        _body,
        grid=grid,
        in_specs=[
            pl.BlockSpec(memory_space=pltpu.SMEM),
            pl.BlockSpec((rows, D), lambda i: (i, 0)),
            pl.BlockSpec((D, m), lambda i: (0, 0)),
            pl.BlockSpec((1, m), lambda i: (0, 0)),
            pl.BlockSpec((1, D), lambda i: (0, 0)),
        ],
        out_specs=[
            pl.BlockSpec((rows, m), lambda i: (i, 0)),
            pl.BlockSpec((rows, D), lambda i: (i, 0)),
            pl.BlockSpec((1, 1), lambda i: (0, 0)),
            pl.BlockSpec((rows, m), lambda i: (i, 0)),
        ],
        out_shape=[
            jax.ShapeDtypeStruct((B, m), jnp.float32),
            jax.ShapeDtypeStruct((B, D), jnp.float32),
            jax.ShapeDtypeStruct((1, 1), jnp.float32),
            jax.ShapeDtypeStruct((B, m), jnp.float32),
        ],
    )(k_arr, x, W, b1, b2)
    return (enc, dec, (nnz[0, 0] / B).astype(jnp.float32), res)


# speedup vs baseline: 1.4698x; 1.4698x over previous
import jax
import jax.numpy as jnp
from jax import lax
from jax.experimental import pallas as pl
from jax.experimental.pallas import tpu as pltpu

_ROWS = 256
_SPLIT = 2


def _row_threshold(a, k):
    rows = a.shape[0]
    kf = k.astype(jnp.float32)
    parts = [a[i * (rows // _SPLIT):(i + 1) * (rows // _SPLIT)]
             for i in range(_SPLIT)]

    def bit_step(i, vs):
        bit = jnp.left_shift(jnp.uint32(1), jnp.uint32(30) - i.astype(jnp.uint32))
        out = []
        for ap, v in zip(parts, vs):
            t = v | bit
            tf = lax.bitcast_convert_type(t, jnp.float32)
            cnt = jnp.sum(jnp.where(ap >= tf, 1.0, 0.0), axis=1, keepdims=True)
            out.append(jnp.where(cnt > kf, t, v))
        return tuple(out)

    v0 = tuple(jnp.zeros((rows // _SPLIT, 1), jnp.uint32) for _ in range(_SPLIT))
    vs = lax.fori_loop(0, 31, bit_step, v0, unroll=4)
    v = jnp.concatenate(vs, axis=0)
    return lax.bitcast_convert_type(v, jnp.float32)


def _body(k_ref, x_ref, w_ref, b1_ref, b2_ref, enc_ref, dec_ref, nnz_ref,
          res_ref):
    enc = jnp.dot(x_ref[...], w_ref[...],
                  preferred_element_type=jnp.float32,
                  precision=lax.Precision.DEFAULT) + b1_ref[...]
    enc_ref[...] = enc
    a = jnp.abs(enc)
    k = k_ref[0]

    thr = _row_threshold(a, k)
    keep = a > thr
    res = jnp.where(keep, enc, 0.0)
    res_ref[...] = res

    @pl.when(pl.program_id(0) == 0)
    def _init():
        nnz_ref[...] = jnp.zeros_like(nnz_ref)

    cnt2 = jnp.sum(jnp.where(keep, 1.0, 0.0), axis=1, keepdims=True)
    nnz_ref[...] += jnp.sum(cnt2, axis=0, keepdims=True)

    dec = lax.dot_general(res, w_ref[...], (((1,), (1,)), ((), ())),
                          preferred_element_type=jnp.float32,
                          precision=lax.Precision.DEFAULT) + b2_ref[...]
    dec_ref[...] = dec


def kernel(x, K, W, b1, b2):
    B, D = x.shape
    m = W.shape[1]
    rows = _ROWS if B % _ROWS == 0 else B
    grid = (B // rows,)
    k_arr = jnp.asarray(K, jnp.int32).reshape(1)
    enc, dec, nnz, res = pl.pallas_call(
        _body,
        grid=grid,
        in_specs=[
            pl.BlockSpec(memory_space=pltpu.SMEM),
            pl.BlockSpec((rows, D), lambda i: (i, 0)),
            pl.BlockSpec((D, m), lambda i: (0, 0)),
            pl.BlockSpec((1, m), lambda i: (0, 0)),
            pl.BlockSpec((1, D), lambda i: (0, 0)),
        ],
        out_specs=[
            pl.BlockSpec((rows, m), lambda i: (i, 0)),
            pl.BlockSpec((rows, D), lambda i: (i, 0)),
            pl.BlockSpec((1, 1), lambda i: (0, 0)),
            pl.BlockSpec((rows, m), lambda i: (i, 0)),
        ],
        out_shape=[
            jax.ShapeDtypeStruct((B, m), jnp.float32),
            jax.ShapeDtypeStruct((B, D), jnp.float32),
            jax.ShapeDtypeStruct((1, 1), jnp.float32),
            jax.ShapeDtypeStruct((B, m), jnp.float32),
        ],
    )(k_arr, x, W, b1, b2)
    return (enc, dec, (nnz[0, 0] / B).astype(jnp.float32), res)


# speedup vs baseline: 1.4937x; 1.0163x over previous
import jax
import jax.numpy as jnp
from jax import lax
from jax.experimental import pallas as pl
from jax.experimental.pallas import tpu as pltpu

_ROWS = 256
_SPLIT = 2


def _row_threshold(a, k):
    rows = a.shape[0]
    kf = k.astype(jnp.float32)
    parts = [a[i * (rows // _SPLIT):(i + 1) * (rows // _SPLIT)]
             for i in range(_SPLIT)]

    def bit_step(i, vs):
        bit = jnp.left_shift(jnp.uint32(1), jnp.uint32(30) - i.astype(jnp.uint32))
        out = []
        for ap, v in zip(parts, vs):
            t = v | bit
            tf = lax.bitcast_convert_type(t, jnp.float32)
            cnt = jnp.sum(jnp.where(ap >= tf, 1.0, 0.0), axis=1, keepdims=True)
            out.append(jnp.where(cnt > kf, t, v))
        return tuple(out)

    v0 = tuple(jnp.zeros((rows // _SPLIT, 1), jnp.uint32) for _ in range(_SPLIT))
    vs = lax.fori_loop(0, 31, bit_step, v0, unroll=8)
    v = jnp.concatenate(vs, axis=0)
    return lax.bitcast_convert_type(v, jnp.float32)


def _body(k_ref, x_ref, w_ref, b1_ref, b2_ref, enc_ref, dec_ref, nnz_ref,
          res_ref):
    enc = jnp.dot(x_ref[...], w_ref[...],
                  preferred_element_type=jnp.float32,
                  precision=lax.Precision.DEFAULT) + b1_ref[...]
    enc_ref[...] = enc
    a = jnp.abs(enc)
    k = k_ref[0]

    thr = _row_threshold(a, k)
    keep = a > thr
    res = jnp.where(keep, enc, 0.0)
    res_ref[...] = res

    @pl.when(pl.program_id(0) == 0)
    def _init():
        nnz_ref[...] = jnp.zeros_like(nnz_ref)

    cnt2 = jnp.sum(jnp.where(keep, 1.0, 0.0), axis=1, keepdims=True)
    nnz_ref[...] += jnp.sum(cnt2, axis=0, keepdims=True)

    dec = lax.dot_general(res, w_ref[...], (((1,), (1,)), ((), ())),
                          preferred_element_type=jnp.float32,
                          precision=lax.Precision.DEFAULT) + b2_ref[...]
    dec_ref[...] = dec


def kernel(x, K, W, b1, b2):
    B, D = x.shape
    m = W.shape[1]
    rows = _ROWS if B % _ROWS == 0 else B
    grid = (B // rows,)
    k_arr = jnp.asarray(K, jnp.int32).reshape(1)
    enc, dec, nnz, res = pl.pallas_call(
        _body,
        grid=grid,
        in_specs=[
            pl.BlockSpec(memory_space=pltpu.SMEM),
            pl.BlockSpec((rows, D), lambda i: (i, 0)),
            pl.BlockSpec((D, m), lambda i: (0, 0)),
            pl.BlockSpec((1, m), lambda i: (0, 0)),
            pl.BlockSpec((1, D), lambda i: (0, 0)),
        ],
        out_specs=[
            pl.BlockSpec((rows, m), lambda i: (i, 0)),
            pl.BlockSpec((rows, D), lambda i: (i, 0)),
            pl.BlockSpec((1, 1), lambda i: (0, 0)),
            pl.BlockSpec((rows, m), lambda i: (i, 0)),
        ],
        out_shape=[
            jax.ShapeDtypeStruct((B, m), jnp.float32),
            jax.ShapeDtypeStruct((B, D), jnp.float32),
            jax.ShapeDtypeStruct((1, 1), jnp.float32),
            jax.ShapeDtypeStruct((B, m), jnp.float32),
        ],
    )(k_arr, x, W, b1, b2)
    return (enc, dec, (nnz[0, 0] / B).astype(jnp.float32), res)


# speedup vs baseline: 1.5469x; 1.0356x over previous
import jax
import jax.numpy as jnp
from jax import lax
from jax.experimental import pallas as pl
from jax.experimental.pallas import tpu as pltpu

_ROWS = 256
_SPLIT = 2


def _row_threshold(a, k):
    rows = a.shape[0]
    kf = k.astype(jnp.float32)
    parts = [a[i * (rows // _SPLIT):(i + 1) * (rows // _SPLIT)]
             for i in range(_SPLIT)]

    def bit_step(i, vs):
        bit = jnp.left_shift(jnp.uint32(1), jnp.uint32(30) - i.astype(jnp.uint32))
        out = []
        for ap, v in zip(parts, vs):
            t = v | bit
            tf = lax.bitcast_convert_type(t, jnp.float32)
            cnt = jnp.sum(jnp.where(ap >= tf, 1.0, 0.0), axis=1, keepdims=True)
            out.append(jnp.where(cnt > kf, t, v))
        return tuple(out)

    v0 = tuple(jnp.zeros((rows // _SPLIT, 1), jnp.uint32) for _ in range(_SPLIT))
    vs = lax.fori_loop(0, 31, bit_step, v0, unroll=16)
    v = jnp.concatenate(vs, axis=0)
    return lax.bitcast_convert_type(v, jnp.float32)


def _body(k_ref, x_ref, w_ref, b1_ref, b2_ref, enc_ref, dec_ref, nnz_ref,
          res_ref):
    enc = jnp.dot(x_ref[...], w_ref[...],
                  preferred_element_type=jnp.float32,
                  precision=lax.Precision.DEFAULT) + b1_ref[...]
    enc_ref[...] = enc
    a = jnp.abs(enc)
    k = k_ref[0]

    thr = _row_threshold(a, k)
    keep = a > thr
    res = jnp.where(keep, enc, 0.0)
    res_ref[...] = res

    @pl.when(pl.program_id(0) == 0)
    def _init():
        nnz_ref[...] = jnp.zeros_like(nnz_ref)

    cnt2 = jnp.sum(jnp.where(keep, 1.0, 0.0), axis=1, keepdims=True)
    nnz_ref[...] += jnp.sum(cnt2, axis=0, keepdims=True)

    dec = lax.dot_general(res, w_ref[...], (((1,), (1,)), ((), ())),
                          preferred_element_type=jnp.float32,
                          precision=lax.Precision.DEFAULT) + b2_ref[...]
    dec_ref[...] = dec


def kernel(x, K, W, b1, b2):
    B, D = x.shape
    m = W.shape[1]
    rows = _ROWS if B % _ROWS == 0 else B
    grid = (B // rows,)
    k_arr = jnp.asarray(K, jnp.int32).reshape(1)
    enc, dec, nnz, res = pl.pallas_call(
        _body,
        grid=grid,
        in_specs=[
            pl.BlockSpec(memory_space=pltpu.SMEM),
            pl.BlockSpec((rows, D), lambda i: (i, 0)),
            pl.BlockSpec((D, m), lambda i: (0, 0)),
            pl.BlockSpec((1, m), lambda i: (0, 0)),
            pl.BlockSpec((1, D), lambda i: (0, 0)),
        ],
        out_specs=[
            pl.BlockSpec((rows, m), lambda i: (i, 0)),
            pl.BlockSpec((rows, D), lambda i: (i, 0)),
            pl.BlockSpec((1, 1), lambda i: (0, 0)),
            pl.BlockSpec((rows, m), lambda i: (i, 0)),
        ],
        out_shape=[
            jax.ShapeDtypeStruct((B, m), jnp.float32),
            jax.ShapeDtypeStruct((B, D), jnp.float32),
            jax.ShapeDtypeStruct((1, 1), jnp.float32),
            jax.ShapeDtypeStruct((B, m), jnp.float32),
        ],
    )(k_arr, x, W, b1, b2)
    return (enc, dec, (nnz[0, 0] / B).astype(jnp.float32), res)


# speedup vs baseline: 1.5472x; 1.0002x over previous
import jax
import jax.numpy as jnp
from jax import lax
from jax.experimental import pallas as pl
from jax.experimental.pallas import tpu as pltpu

_ROWS = 256
_SPLIT = 2


def _row_threshold(a, k):
    rows = a.shape[0]
    kf = k.astype(jnp.float32)
    parts = [a[i * (rows // _SPLIT):(i + 1) * (rows // _SPLIT)]
             for i in range(_SPLIT)]

    def bit_step(i, vs):
        bit = jnp.left_shift(jnp.uint32(1), jnp.uint32(30) - i.astype(jnp.uint32))
        out = []
        for ap, v in zip(parts, vs):
            t = v | bit
            tf = lax.bitcast_convert_type(t, jnp.float32)
            cnt = jnp.sum(jnp.where(ap >= tf, 1.0, 0.0), axis=1, keepdims=True)
            out.append(jnp.where(cnt > kf, t, v))
        return tuple(out)

    v0 = tuple(jnp.zeros((rows // _SPLIT, 1), jnp.uint32) for _ in range(_SPLIT))
    vs = lax.fori_loop(0, 31, bit_step, v0, unroll=31)
    v = jnp.concatenate(vs, axis=0)
    return lax.bitcast_convert_type(v, jnp.float32)


def _body(k_ref, x_ref, w_ref, b1_ref, b2_ref, enc_ref, dec_ref, nnz_ref,
          res_ref):
    enc = jnp.dot(x_ref[...], w_ref[...],
                  preferred_element_type=jnp.float32,
                  precision=lax.Precision.DEFAULT) + b1_ref[...]
    enc_ref[...] = enc
    a = jnp.abs(enc)
    k = k_ref[0]

    thr = _row_threshold(a, k)
    keep = a > thr
    res = jnp.where(keep, enc, 0.0)
    res_ref[...] = res

    @pl.when(pl.program_id(0) == 0)
    def _init():
        nnz_ref[...] = jnp.zeros_like(nnz_ref)

    cnt2 = jnp.sum(jnp.where(keep, 1.0, 0.0), axis=1, keepdims=True)
    nnz_ref[...] += jnp.sum(cnt2, axis=0, keepdims=True)

    dec = lax.dot_general(res, w_ref[...], (((1,), (1,)), ((), ())),
                          preferred_element_type=jnp.float32,
                          precision=lax.Precision.DEFAULT) + b2_ref[...]
    dec_ref[...] = dec


def kernel(x, K, W, b1, b2):
    B, D = x.shape
    m = W.shape[1]
    rows = _ROWS if B % _ROWS == 0 else B
    grid = (B // rows,)
    k_arr = jnp.asarray(K, jnp.int32).reshape(1)
    enc, dec, nnz, res = pl.pallas_call(
        _body,
        grid=grid,
        in_specs=[
            pl.BlockSpec(memory_space=pltpu.SMEM),
            pl.BlockSpec((rows, D), lambda i: (i, 0)),
            pl.BlockSpec((D, m), lambda i: (0, 0)),
            pl.BlockSpec((1, m), lambda i: (0, 0)),
            pl.BlockSpec((1, D), lambda i: (0, 0)),
        ],
        out_specs=[
            pl.BlockSpec((rows, m), lambda i: (i, 0)),
            pl.BlockSpec((rows, D), lambda i: (i, 0)),
            pl.BlockSpec((1, 1), lambda i: (0, 0)),
            pl.BlockSpec((rows, m), lambda i: (i, 0)),
        ],
        out_shape=[
            jax.ShapeDtypeStruct((B, m), jnp.float32),
            jax.ShapeDtypeStruct((B, D), jnp.float32),
            jax.ShapeDtypeStruct((1, 1), jnp.float32),
            jax.ShapeDtypeStruct((B, m), jnp.float32),
        ],
    )(k_arr, x, W, b1, b2)
    return (enc, dec, (nnz[0, 0] / B).astype(jnp.float32), res)


# speedup vs baseline: 1.5479x; 1.0005x over previous
import jax
import jax.numpy as jnp
from jax import lax
from jax.experimental import pallas as pl
from jax.experimental.pallas import tpu as pltpu

_ROWS = 256
_SPLIT = 4


def _row_threshold(a, k):
    rows = a.shape[0]
    kf = k.astype(jnp.float32)
    parts = [a[i * (rows // _SPLIT):(i + 1) * (rows // _SPLIT)]
             for i in range(_SPLIT)]

    def bit_step(i, vs):
        bit = jnp.left_shift(jnp.uint32(1), jnp.uint32(30) - i.astype(jnp.uint32))
        out = []
        for ap, v in zip(parts, vs):
            t = v | bit
            tf = lax.bitcast_convert_type(t, jnp.float32)
            cnt = jnp.sum(jnp.where(ap >= tf, 1.0, 0.0), axis=1, keepdims=True)
            out.append(jnp.where(cnt > kf, t, v))
        return tuple(out)

    v0 = tuple(jnp.zeros((rows // _SPLIT, 1), jnp.uint32) for _ in range(_SPLIT))
    vs = lax.fori_loop(0, 31, bit_step, v0, unroll=16)
    v = jnp.concatenate(vs, axis=0)
    return lax.bitcast_convert_type(v, jnp.float32)


def _body(k_ref, x_ref, w_ref, b1_ref, b2_ref, enc_ref, dec_ref, nnz_ref,
          res_ref):
    enc = jnp.dot(x_ref[...], w_ref[...],
                  preferred_element_type=jnp.float32,
                  precision=lax.Precision.DEFAULT) + b1_ref[...]
    enc_ref[...] = enc
    a = jnp.abs(enc)
    k = k_ref[0]

    thr = _row_threshold(a, k)
    keep = a > thr
    res = jnp.where(keep, enc, 0.0)
    res_ref[...] = res

    @pl.when(pl.program_id(0) == 0)
    def _init():
        nnz_ref[...] = jnp.zeros_like(nnz_ref)

    cnt2 = jnp.sum(jnp.where(keep, 1.0, 0.0), axis=1, keepdims=True)
    nnz_ref[...] += jnp.sum(cnt2, axis=0, keepdims=True)

    dec = lax.dot_general(res, w_ref[...], (((1,), (1,)), ((), ())),
                          preferred_element_type=jnp.float32,
                          precision=lax.Precision.DEFAULT) + b2_ref[...]
    dec_ref[...] = dec


def kernel(x, K, W, b1, b2):
    B, D = x.shape
    m = W.shape[1]
    rows = _ROWS if B % _ROWS == 0 else B
    grid = (B // rows,)
    k_arr = jnp.asarray(K, jnp.int32).reshape(1)
    enc, dec, nnz, res = pl.pallas_call(
        _body,
        grid=grid,
        in_specs=[
            pl.BlockSpec(memory_space=pltpu.SMEM),
            pl.BlockSpec((rows, D), lambda i: (i, 0)),
            pl.BlockSpec((D, m), lambda i: (0, 0)),
            pl.BlockSpec((1, m), lambda i: (0, 0)),
            pl.BlockSpec((1, D), lambda i: (0, 0)),
        ],
        out_specs=[
            pl.BlockSpec((rows, m), lambda i: (i, 0)),
            pl.BlockSpec((rows, D), lambda i: (i, 0)),
            pl.BlockSpec((1, 1), lambda i: (0, 0)),
            pl.BlockSpec((rows, m), lambda i: (i, 0)),
        ],
        out_shape=[
            jax.ShapeDtypeStruct((B, m), jnp.float32),
            jax.ShapeDtypeStruct((B, D), jnp.float32),
            jax.ShapeDtypeStruct((1, 1), jnp.float32),
            jax.ShapeDtypeStruct((B, m), jnp.float32),
        ],
    )(k_arr, x, W, b1, b2)
    return (enc, dec, (nnz[0, 0] / B).astype(jnp.float32), res)
